# baseline (speedup 1.0000x reference)
import functools

import jax
import jax.numpy as jnp
from jax import lax
from jax.experimental import pallas as pl
from jax.experimental.pallas import tpu as pltpu
from jax.experimental.pallas import tpu_sc as plsc

_NC, _NS = 2, 16
_NW = _NC * _NS

_BATCH = 16384
_H = 50
_D = 64
_CT = _BATCH // 128
_CPW = _CT // _NW
_NBUF = 4


@functools.partial(
    pl.kernel,
    out_type=jax.ShapeDtypeStruct((_H, _D // 8, _CT, 8, 128), jnp.float32),
    mesh=plsc.VectorSubcoreMesh(core_axis_name="c", subcore_axis_name="s"),
    scratch_types=[
        pltpu.VMEM((_H, 128 * _CPW), jnp.int32),
        pltpu.VMEM((_NBUF, 128, _D), jnp.float32),
        pltpu.VMEM((_NBUF, _D // 8, 8, 128), jnp.float32),
        pltpu.SemaphoreType.DMA((_NBUF,)),
        pltpu.SemaphoreType.DMA((_NBUF,)),
    ],
    compiler_params=pltpu.CompilerParams(
        use_tc_tiling_on_sc=False, needs_layout_passes=False
    ),
)
def _lookup(table_hbm, idxt_hbm, out_hbm, idx_v, rows_v, trans_v, gsem, ssem):
    wid = lax.axis_index("s") * _NC + lax.axis_index("c")
    pltpu.sync_copy(idxt_hbm.at[:, pl.ds(wid * 128 * _CPW, 128 * _CPW)], idx_v)

    lanes = lax.iota(jnp.int32, 16)
    c_idx = [c0 + lanes for c0 in range(0, 128, 16)]

    def gather_start(slot, h):
        return pltpu.async_copy(
            table_hbm.at[idx_v.at[h, pl.ds(128 * slot, 128)]],
            rows_v.at[slot],
            gsem.at[slot],
        )

    for b in range(_NBUF - 1):
        gather_start(b, 0)

    @pl.loop(0, _H)
    def _per_h(h):
        for j in range(_CPW):
            pltpu.make_async_copy(
                table_hbm.at[idx_v.at[h, pl.ds(128 * j, 128)]],
                rows_v.at[j],
                gsem.at[j],
            ).wait()
            fj = (j + _NBUF - 1) % _NBUF
            fh = h + (j + _NBUF - 1) // _NBUF

            @pl.when(fh < _H)
            def _():
                gather_start(fj, fh)

            @pl.when(h >= 1)
            def _():
                pltpu.make_async_copy(
                    trans_v.at[j], out_hbm.at[0, :, 0], ssem.at[j]
                ).wait()

            for r8 in range(_D // 8):
                for r in range(8):
                    d_vec = jnp.full((16,), r8 * 8 + r, jnp.int32)
                    for g, ci in enumerate(c_idx):
                        vals = plsc.load_gather(rows_v.at[j], [ci, d_vec])
                        trans_v[j, r8, r, pl.ds(16 * g, 16)] = vals

            pltpu.async_copy(
                trans_v.at[j], out_hbm.at[h, :, wid * _CPW + j], ssem.at[j]
            )

    for j in range(_NBUF):
        pltpu.make_async_copy(
            trans_v.at[j], out_hbm.at[0, :, 0], ssem.at[j]
        ).wait()


def kernel(table, indices):
    out5 = _lookup(table, indices.T)
    return out5.transpose(2, 4, 0, 1, 3).reshape(_BATCH, _H, _D)


# speedup vs baseline: 1.5202x; 1.5202x over previous
import functools

import jax
import jax.numpy as jnp
from jax import lax
from jax.experimental import pallas as pl
from jax.experimental.pallas import tpu as pltpu
from jax.experimental.pallas import tpu_sc as plsc

_NC, _NS = 2, 16
_NW = _NC * _NS

_BATCH = 16384
_H = 50
_D = 64
_CT = _BATCH // 128
_CPW = _CT // _NW
_NBUF = 4


@functools.partial(
    pl.kernel,
    out_type=jax.ShapeDtypeStruct((_H, _D // 8, _CT, 8, 128), jnp.float32),
    mesh=plsc.VectorSubcoreMesh(core_axis_name="c", subcore_axis_name="s"),
    scratch_types=[
        pltpu.VMEM((_H, 128 * _CPW), jnp.int32),
        pltpu.VMEM((_NBUF, 128, _D), jnp.float32),
        pltpu.VMEM((_NBUF, _D // 8, 8, 128), jnp.float32),
        pltpu.SemaphoreType.DMA((_NBUF,)),
        pltpu.SemaphoreType.DMA((_NBUF,)),
    ],
    compiler_params=pltpu.CompilerParams(
        use_tc_tiling_on_sc=False, needs_layout_passes=False
    ),
)
def _lookup(table_hbm, idxt_hbm, out_hbm, idx_v, rows_v, trans_v, gsem, ssem):
    wid = lax.axis_index("s") * _NC + lax.axis_index("c")
    pltpu.sync_copy(idxt_hbm.at[:, pl.ds(wid * 128 * _CPW, 128 * _CPW)], idx_v)

    lanes = lax.iota(jnp.int32, 16)
    c_idx = [c0 + lanes for c0 in range(0, 128, 16)]

    def gather_start(slot, h):
        return pltpu.async_copy(
            table_hbm.at[idx_v.at[h, pl.ds(128 * slot, 128)]],
            rows_v.at[slot],
            gsem.at[slot],
        )

    for b in range(_NBUF - 1):
        gather_start(b, 0)

    @pl.loop(0, _H)
    def _per_h(h):
        for j in range(_CPW):
            pltpu.make_async_copy(
                table_hbm.at[idx_v.at[h, pl.ds(128 * j, 128)]],
                rows_v.at[j],
                gsem.at[j],
            ).wait()
            fj = (j + _NBUF - 1) % _NBUF
            fh = h + (j + _NBUF - 1) // _NBUF

            @pl.when(fh < _H)
            def _():
                gather_start(fj, fh)

            @pl.when(h >= 1)
            def _():
                pltpu.make_async_copy(
                    trans_v.at[j], out_hbm.at[0, :, 0], ssem.at[j]
                ).wait()

            @plsc.parallel_loop(0, _D, unroll=8)
            def _per_d(d):
                r8 = d >> 3
                r = d & 7
                d_vec = jnp.full((16,), d, jnp.int32)
                for g, ci in enumerate(c_idx):
                    vals = plsc.load_gather(rows_v.at[j], [ci, d_vec])
                    trans_v[j, r8, r, pl.ds(16 * g, 16)] = vals

            pltpu.async_copy(
                trans_v.at[j], out_hbm.at[h, :, wid * _CPW + j], ssem.at[j]
            )

    for j in range(_NBUF):
        pltpu.make_async_copy(
            trans_v.at[j], out_hbm.at[0, :, 0], ssem.at[j]
        ).wait()


def kernel(table, indices):
    out5 = _lookup(table, indices.T)
    return out5.transpose(2, 4, 0, 1, 3).reshape(_BATCH, _H, _D)


# speedup vs baseline: 1.5264x; 1.0041x over previous
import functools

import jax
import jax.numpy as jnp
from jax import lax
from jax.experimental import pallas as pl
from jax.experimental.pallas import tpu as pltpu
from jax.experimental.pallas import tpu_sc as plsc

_NC, _NS = 2, 16
_NW = _NC * _NS

_BATCH = 16384
_H = 50
_D = 64
_CT = _BATCH // 128
_CPW = _CT // _NW
_NBUF = 4


@functools.partial(
    pl.kernel,
    out_type=jax.ShapeDtypeStruct((_H, _D // 8, _CT, 8, 128), jnp.float32),
    mesh=plsc.VectorSubcoreMesh(core_axis_name="c", subcore_axis_name="s"),
    scratch_types=[
        pltpu.VMEM((_H, 128 * _CPW), jnp.int32),
        pltpu.VMEM((_NBUF, 128, _D), jnp.float32),
        pltpu.VMEM((_NBUF, _D // 8, 8, 128), jnp.float32),
        pltpu.SemaphoreType.DMA((_NBUF,)),
        pltpu.SemaphoreType.DMA((_NBUF,)),
    ],
    compiler_params=pltpu.CompilerParams(
        use_tc_tiling_on_sc=False, needs_layout_passes=False
    ),
)
def _lookup(table_hbm, idxt_hbm, out_hbm, idx_v, rows_v, trans_v, gsem, ssem):
    wid = lax.axis_index("s") * _NC + lax.axis_index("c")
    pltpu.sync_copy(idxt_hbm.at[:, pl.ds(wid * 128 * _CPW, 128 * _CPW)], idx_v)

    lanes = lax.iota(jnp.int32, 16)
    c_idx = [c0 + lanes for c0 in range(0, 128, 16)]

    def gather_start(slot, h):
        return pltpu.async_copy(
            table_hbm.at[idx_v.at[h, pl.ds(128 * slot, 128)]],
            rows_v.at[slot],
            gsem.at[slot],
        )

    for b in range(_NBUF - 1):
        gather_start(b, 0)

    @pl.loop(0, _H)
    def _per_h(h):
        for j in range(_CPW):
            pltpu.make_async_copy(
                table_hbm.at[idx_v.at[h, pl.ds(128 * j, 128)]],
                rows_v.at[j],
                gsem.at[j],
            ).wait()
            fj = (j + _NBUF - 1) % _NBUF
            fh = h + (j + _NBUF - 1) // _NBUF

            @pl.when(fh < _H)
            def _():
                gather_start(fj, fh)

            @pl.when(h >= 1)
            def _():
                pltpu.make_async_copy(
                    trans_v.at[j], out_hbm.at[0, :, 0], ssem.at[j]
                ).wait()

            @plsc.parallel_loop(0, _D, unroll=16)
            def _per_d(d):
                r8 = d >> 3
                r = d & 7
                d_vec = jnp.full((16,), d, jnp.int32)
                for g, ci in enumerate(c_idx):
                    vals = plsc.load_gather(rows_v.at[j], [ci, d_vec])
                    trans_v[j, r8, r, pl.ds(16 * g, 16)] = vals

            pltpu.async_copy(
                trans_v.at[j], out_hbm.at[h, :, wid * _CPW + j], ssem.at[j]
            )

    for j in range(_NBUF):
        pltpu.make_async_copy(
            trans_v.at[j], out_hbm.at[0, :, 0], ssem.at[j]
        ).wait()


def kernel(table, indices):
    out5 = _lookup(table, indices.T)
    return out5.transpose(2, 4, 0, 1, 3).reshape(_BATCH, _H, _D)


# speedup vs baseline: 1.6792x; 1.1002x over previous
import functools

import jax
import jax.numpy as jnp
from jax import lax
from jax.experimental import pallas as pl
from jax.experimental.pallas import tpu as pltpu
from jax.experimental.pallas import tpu_sc as plsc

_NC, _NS = 2, 16
_NW = _NC * _NS

_BATCH = 16384
_H = 50
_D = 64
_CT = _BATCH // 128
_CPW = _CT // _NW
_NBUF = 4


@functools.partial(
    pl.kernel,
    out_type=jax.ShapeDtypeStruct((_H, _D // 8, _CT, 8, 128), jnp.float32),
    mesh=plsc.VectorSubcoreMesh(core_axis_name="c", subcore_axis_name="s"),
    scratch_types=[
        pltpu.VMEM((_H, 128 * _CPW), jnp.int32),
        pltpu.VMEM((_NBUF, 128, _D), jnp.float32),
        pltpu.VMEM((_NBUF, _D // 8, 8, 128), jnp.float32),
        pltpu.SemaphoreType.DMA((_NBUF,)),
        pltpu.SemaphoreType.DMA((_NBUF,)),
    ],
    compiler_params=pltpu.CompilerParams(
        use_tc_tiling_on_sc=False, needs_layout_passes=False
    ),
)
def _lookup(table_hbm, idxt_hbm, out_hbm, idx_v, rows_v, trans_v, gsem, ssem):
    wid = lax.axis_index("s") * _NC + lax.axis_index("c")
    pltpu.sync_copy(idxt_hbm.at[:, pl.ds(wid * 128 * _CPW, 128 * _CPW)], idx_v)

    lanes = lax.iota(jnp.int32, 16)
    c_idx = [c0 + lanes for c0 in range(0, 128, 16)]
    diag = [(lanes + k) & 15 for k in range(16)]

    def gather_start(slot, h):
        return pltpu.async_copy(
            table_hbm.at[idx_v.at[h, pl.ds(128 * slot, 128)]],
            rows_v.at[slot],
            gsem.at[slot],
        )

    for b in range(_NBUF - 1):
        gather_start(b, 0)

    @pl.loop(0, _H)
    def _per_h(h):
        for j in range(_CPW):
            pltpu.make_async_copy(
                table_hbm.at[idx_v.at[h, pl.ds(128 * j, 128)]],
                rows_v.at[j],
                gsem.at[j],
            ).wait()
            fj = (j + _NBUF - 1) % _NBUF
            fh = h + (j + _NBUF - 1) // _NBUF

            @pl.when(fh < _H)
            def _():
                gather_start(fj, fh)

            @pl.when(h >= 1)
            def _():
                pltpu.make_async_copy(
                    trans_v.at[j], out_hbm.at[0, :, 0], ssem.at[j]
                ).wait()

            @plsc.parallel_loop(0, _D, step=16)
            def _per_d0(d0):
                for k in range(16):
                    d_idx = diag[k] + d0
                    r8_idx = d_idx >> 3
                    r_idx = d_idx & 7
                    for g, ci in enumerate(c_idx):
                        vals = plsc.load_gather(rows_v.at[j], [ci, d_idx])
                        plsc.store_scatter(
                            trans_v.at[j], [r8_idx, r_idx, ci], vals
                        )

            pltpu.async_copy(
                trans_v.at[j], out_hbm.at[h, :, wid * _CPW + j], ssem.at[j]
            )

    for j in range(_NBUF):
        pltpu.make_async_copy(
            trans_v.at[j], out_hbm.at[0, :, 0], ssem.at[j]
        ).wait()


def kernel(table, indices):
    out5 = _lookup(table, indices.T)
    return out5.transpose(2, 4, 0, 1, 3).reshape(_BATCH, _H, _D)


# speedup vs baseline: 2.0946x; 1.2474x over previous
import functools

import jax
import jax.numpy as jnp
from jax import lax
from jax.experimental import pallas as pl
from jax.experimental.pallas import tpu as pltpu
from jax.experimental.pallas import tpu_sc as plsc

_NC, _NS = 2, 16
_NW = _NC * _NS

_BATCH = 16384
_H = 50
_D = 64
_CT = _BATCH // 128
_CPW = _CT // _NW
_NBUF = 4


@functools.partial(
    pl.kernel,
    out_type=jax.ShapeDtypeStruct((_H, _D // 8, _CT, 8, 128), jnp.float32),
    mesh=plsc.VectorSubcoreMesh(core_axis_name="c", subcore_axis_name="s"),
    scratch_types=[
        pltpu.VMEM((_H, 128 * _CPW), jnp.int32),
        pltpu.VMEM((_NBUF, 128, _D), jnp.float32),
        pltpu.VMEM((_NBUF, _D // 8, 8, 128), jnp.float32),
        pltpu.SemaphoreType.DMA((_NBUF,)),
        pltpu.SemaphoreType.DMA((_NBUF,)),
    ],
    compiler_params=pltpu.CompilerParams(
        use_tc_tiling_on_sc=False, needs_layout_passes=False
    ),
)
def _lookup(table_hbm, idxt_hbm, out_hbm, idx_v, rows_v, trans_v, gsem, ssem):
    wid = lax.axis_index("s") * _NC + lax.axis_index("c")
    pltpu.sync_copy(idxt_hbm.at[:, pl.ds(wid * 128 * _CPW, 128 * _CPW)], idx_v)

    lanes = lax.iota(jnp.int32, 16)
    c_idx = [c0 + lanes for c0 in range(0, 128, 16)]
    diag = [(lanes + k) & 15 for k in range(16)]
    diag_h = [d >> 3 for d in diag]
    diag_l = [d & 7 for d in diag]

    def gather_start(slot, h):
        return pltpu.async_copy(
            table_hbm.at[idx_v.at[h, pl.ds(128 * slot, 128)]],
            rows_v.at[slot],
            gsem.at[slot],
        )

    for b in range(_NBUF - 1):
        gather_start(b, 0)

    @pl.loop(0, _H)
    def _per_h(h):
        for j in range(_CPW):
            pltpu.make_async_copy(
                table_hbm.at[idx_v.at[h, pl.ds(128 * j, 128)]],
                rows_v.at[j],
                gsem.at[j],
            ).wait()
            fj = (j + _NBUF - 1) % _NBUF
            fh = h + (j + _NBUF - 1) // _NBUF

            @pl.when(fh < _H)
            def _():
                gather_start(fj, fh)

            @pl.when(h >= 1)
            def _():
                pltpu.make_async_copy(
                    trans_v.at[j], out_hbm.at[0, :, 0], ssem.at[j]
                ).wait()

            @plsc.parallel_loop(0, _D, step=16, unroll=2)
            def _per_d0(d0):
                d0h = d0 >> 3
                for k in range(16):
                    d_idx = diag[k] + d0
                    r8_idx = diag_h[k] + d0h
                    for g, ci in enumerate(c_idx):
                        vals = plsc.load_gather(rows_v.at[j], [ci, d_idx])
                        plsc.store_scatter(
                            trans_v.at[j], [r8_idx, diag_l[k], ci], vals
                        )

            pltpu.async_copy(
                trans_v.at[j], out_hbm.at[h, :, wid * _CPW + j], ssem.at[j]
            )

    for j in range(_NBUF):
        pltpu.make_async_copy(
            trans_v.at[j], out_hbm.at[0, :, 0], ssem.at[j]
        ).wait()


def kernel(table, indices):
    out5 = _lookup(table, indices.T)
    return out5.transpose(2, 4, 0, 1, 3).reshape(_BATCH, _H, _D)
